# baseline (speedup 1.0000x reference)
import functools

import jax
import jax.numpy as jnp
from jax import lax
from jax.experimental import pallas as pl
from jax.experimental.pallas import tpu as pltpu
from jax.experimental.pallas import tpu_sc as plsc

_NUM_BINS = 32
_NCH = 3
_L = 16
_NSEG = _NCH * _NUM_BINS
_NREP = 8
_REGION = _NSEG * _L
_ACC_SIZE = (_NREP // 2) * _REGION
_SHIFT = 19
_QBIAS = 524288
_QMASK = _QBIAS - 1

_NUNITS = 96
_UNIT = 65536
_ROWS = 32
_CHUNK = _ROWS * 512
_CHUNKS_PER_UNIT = _UNIT // _CHUNK
_NW = 32
_UNITS_PER_W = _NUNITS // _NW


def _sc_body(x_hbm, p_hbm, t_hbm, sums_out, cnts_out,
             xb0, xb1, pb0, pb1, tb0, tb1,
             acc_a, acc_b, row_s, row_c, sem0, sem1):
    wid = lax.axis_index("s") * 2 + lax.axis_index("c")

    izeros = jnp.zeros((_L,), jnp.int32)
    lane_iota = lax.iota(jnp.int32, _L)

    def _zero(k, _):
        acc_a[pl.ds(k * _L, _L)] = izeros
        acc_b[pl.ds(k * _L, _L)] = izeros
        return 0
    lax.fori_loop(0, _ACC_SIZE // _L, _zero, 0)

    nsteps = _UNITS_PER_W * _CHUNKS_PER_UNIT
    bufs = ((xb0, pb0, tb0), (xb1, pb1, tb1))
    sems = (sem0, sem1)

    def _issue(step, slot):
        j, cidx = divmod(step, _CHUNKS_PER_UNIT)
        u = wid * _UNITS_PER_W + j
        slab = lax.div(u, 4)
        row0 = lax.rem(u, 4) * (4 * _ROWS) + cidx * _ROWS
        xs, ps, ts = bufs[slot]
        sem = sems[slot]
        return (
            pltpu.async_copy(x_hbm.at[slab, pl.ds(row0, _ROWS)], xs, sem),
            pltpu.async_copy(p_hbm.at[slab, pl.ds(row0, _ROWS)], ps, sem),
            pltpu.async_copy(t_hbm.at[slab, pl.ds(row0, _ROWS)], ts, sem),
        )

    inflight = _issue(0, 0)
    for step in range(nsteps):
        slot = step % 2
        cur = inflight
        if step + 1 < nsteps:
            inflight = _issue(step + 1, 1 - slot)
        for c in cur:
            c.wait()
        j = step // _CHUNKS_PER_UNIT
        u = wid * _UNITS_PER_W + j
        ch = lax.rem(lax.div(u, 4), _NCH)
        laneoff = lane_iota + ch * (_NUM_BINS * _L)
        xs, ps, ts = bufs[slot]

        def _vec(g, _):
            for r in range(_NREP):
                v = g * _NREP + r
                a = (acc_a, acc_b)[r % 2]
                row = lax.div(v, 32)
                col = lax.rem(v, 32) * _L
                x = xs[row, pl.ds(col, _L)]
                d = ps[row, pl.ds(col, _L)] - ts[row, pl.ds(col, _L)]
                bi = (x * jnp.float32(_NUM_BINS)).astype(jnp.int32)
                bi = jnp.minimum(jnp.maximum(bi, 0), _NUM_BINS - 1)
                idx = laneoff + bi * _L + (r // 2) * _REGION
                q = (d * jnp.float32(128.0) + jnp.float32(128.5)).astype(jnp.int32)
                plsc.addupdate_scatter(a, [idx], q + _QBIAS)
            return 0
        lax.fori_loop(0, _CHUNK // (_NREP * _L), _vec, 0, unroll=2)

    giota = lane_iota * _L
    for k in range(_NSEG // _L):
        def _red(m, carry):
            ncnt, qs = carry
            rep = lax.div(m, _L)
            lane = lax.rem(m, _L)
            gidx = giota + (rep * _REGION + k * (_L * _L) + lane)
            g = plsc.load_gather(acc_a, [gidx]) + plsc.load_gather(acc_b, [gidx])
            return (ncnt + lax.shift_right_logical(g, _SHIFT),
                    qs + (g & _QMASK))
        ncnt, qs = lax.fori_loop(0, (_NREP // 2) * _L, _red, (izeros, izeros))
        cntf = ncnt.astype(jnp.float32)
        row_s[pl.ds(k * _L, _L)] = qs.astype(jnp.float32) / jnp.float32(128.0) - cntf
        row_c[pl.ds(k * _L, _L)] = cntf

    pltpu.sync_copy(row_s, sums_out.at[wid])
    pltpu.sync_copy(row_c, cnts_out.at[wid])


_sc_call = functools.partial(
    pl.kernel,
    out_type=(jax.ShapeDtypeStruct((_NW, _NSEG), jnp.float32),
              jax.ShapeDtypeStruct((_NW, _NSEG), jnp.float32)),
    mesh=plsc.VectorSubcoreMesh(core_axis_name="c", subcore_axis_name="s"),
    compiler_params=pltpu.CompilerParams(needs_layout_passes=False),
    scratch_types=(
        pltpu.VMEM((_ROWS, 512), jnp.float32),
        pltpu.VMEM((_ROWS, 512), jnp.float32),
        pltpu.VMEM((_ROWS, 512), jnp.float32),
        pltpu.VMEM((_ROWS, 512), jnp.float32),
        pltpu.VMEM((_ROWS, 512), jnp.float32),
        pltpu.VMEM((_ROWS, 512), jnp.float32),
        pltpu.VMEM((_ACC_SIZE,), jnp.int32),
        pltpu.VMEM((_ACC_SIZE,), jnp.int32),
        pltpu.VMEM((_NSEG,), jnp.float32),
        pltpu.VMEM((_NSEG,), jnp.float32),
        pltpu.SemaphoreType.DMA,
        pltpu.SemaphoreType.DMA,
    ),
)(_sc_body)


def _tc_body(s_ref, c_ref, o_ref):
    s = jnp.sum(s_ref[...], axis=0, keepdims=True)
    c = jnp.sum(c_ref[...], axis=0, keepdims=True)
    nonempty = c > 0.0
    val = jnp.where(nonempty, jnp.abs(s) / jnp.where(nonempty, c, 1.0), 0.0)
    o_ref[...] = jnp.full((1, 1), jnp.sum(val) / jnp.float32(_NSEG))


def kernel(pred, target, input_img):
    x = input_img.reshape(24, 512, 512)
    p = pred.reshape(24, 512, 512)
    t = target.reshape(24, 512, 512)
    sums, cnts = _sc_call(x, p, t)
    loss = pl.pallas_call(
        _tc_body,
        out_shape=jax.ShapeDtypeStruct((1, 1), jnp.float32),
    )(sums, cnts)
    return loss[0, 0]


# speedup vs baseline: 2.9207x; 2.9207x over previous
import functools

import jax
import jax.numpy as jnp
from jax import lax
from jax.experimental import pallas as pl
from jax.experimental.pallas import tpu as pltpu
from jax.experimental.pallas import tpu_sc as plsc

_NUM_BINS = 32
_NCH = 3
_L = 16
_NSEG = _NCH * _NUM_BINS
_ACC_SIZE = _NSEG * _L

_NUNITS = 96
_UNIT = 65536
_ROWS = 32
_CHUNK = _ROWS * 512
_CHUNKS_PER_UNIT = _UNIT // _CHUNK
_NW = 32
_UNITS_PER_W = _NUNITS // _NW


def _sc_body(x_hbm, p_hbm, t_hbm, sums_out, cnts_out,
             xb0, xb1, pb0, pb1, tb0, tb1,
             acc_s, acc_s1, acc_c, acc_c1, row_s, row_c, sem0, sem1):
    wid = lax.axis_index("s") * 2 + lax.axis_index("c")

    zeros = jnp.zeros((_L,), jnp.float32)
    ones = jnp.ones((_L,), jnp.float32)
    lane_iota = lax.iota(jnp.int32, _L)

    def _zero(k, _):
        acc_s[pl.ds(k * _L, _L)] = zeros
        acc_s1[pl.ds(k * _L, _L)] = zeros
        acc_c[pl.ds(k * _L, _L)] = zeros
        acc_c1[pl.ds(k * _L, _L)] = zeros
        return 0
    lax.fori_loop(0, _ACC_SIZE // _L, _zero, 0)

    nsteps = _UNITS_PER_W * _CHUNKS_PER_UNIT
    bufs = ((xb0, pb0, tb0), (xb1, pb1, tb1))
    sems = (sem0, sem1)

    def _issue(step, slot):
        j, cidx = divmod(step, _CHUNKS_PER_UNIT)
        u = wid * _UNITS_PER_W + j
        slab = lax.div(u, 4)
        row0 = lax.rem(u, 4) * (4 * _ROWS) + cidx * _ROWS
        xs, ps, ts = bufs[slot]
        sem = sems[slot]
        return (
            pltpu.async_copy(x_hbm.at[slab, pl.ds(row0, _ROWS)], xs, sem),
            pltpu.async_copy(p_hbm.at[slab, pl.ds(row0, _ROWS)], ps, sem),
            pltpu.async_copy(t_hbm.at[slab, pl.ds(row0, _ROWS)], ts, sem),
        )

    inflight = _issue(0, 0)
    for step in range(nsteps):
        slot = step % 2
        cur = inflight
        if step + 1 < nsteps:
            inflight = _issue(step + 1, 1 - slot)
        for c in cur:
            c.wait()
        j = step // _CHUNKS_PER_UNIT
        u = wid * _UNITS_PER_W + j
        ch = lax.rem(lax.div(u, 4), _NCH)
        laneoff = lane_iota + ch * (_NUM_BINS * _L)
        xs, ps, ts = bufs[slot]

        @plsc.parallel_loop(0, _CHUNK // (2 * _L), 1, unroll=4)
        def _vec(g):
            for r, (a_s, a_c) in enumerate(((acc_s, acc_c), (acc_s1, acc_c1))):
                v = g * 2 + r
                row = lax.div(v, 32)
                col = lax.rem(v, 32) * _L
                x = xs[row, pl.ds(col, _L)]
                d = ps[row, pl.ds(col, _L)] - ts[row, pl.ds(col, _L)]
                bi = (x * jnp.float32(_NUM_BINS)).astype(jnp.int32)
                bi = jnp.minimum(jnp.maximum(bi, 0), _NUM_BINS - 1)
                idx = laneoff + bi * _L
                plsc.addupdate_scatter(a_s, [idx], d)
                plsc.addupdate_scatter(a_c, [idx], ones)

    giota = lane_iota * _L
    for k in range(_NSEG // _L):
        ts = zeros
        tc = zeros
        for lane in range(_L):
            gidx = giota + (k * _L * _L + lane)
            ts = ts + plsc.load_gather(acc_s, [gidx]) + plsc.load_gather(acc_s1, [gidx])
            tc = tc + plsc.load_gather(acc_c, [gidx]) + plsc.load_gather(acc_c1, [gidx])
        row_s[pl.ds(k * _L, _L)] = ts
        row_c[pl.ds(k * _L, _L)] = tc

    pltpu.sync_copy(row_s, sums_out.at[wid])
    pltpu.sync_copy(row_c, cnts_out.at[wid])


_sc_call = functools.partial(
    pl.kernel,
    out_type=(jax.ShapeDtypeStruct((_NW, _NSEG), jnp.float32),
              jax.ShapeDtypeStruct((_NW, _NSEG), jnp.float32)),
    mesh=plsc.VectorSubcoreMesh(core_axis_name="c", subcore_axis_name="s"),
    compiler_params=pltpu.CompilerParams(needs_layout_passes=False),
    scratch_types=(
        pltpu.VMEM((_ROWS, 512), jnp.float32),
        pltpu.VMEM((_ROWS, 512), jnp.float32),
        pltpu.VMEM((_ROWS, 512), jnp.float32),
        pltpu.VMEM((_ROWS, 512), jnp.float32),
        pltpu.VMEM((_ROWS, 512), jnp.float32),
        pltpu.VMEM((_ROWS, 512), jnp.float32),
        pltpu.VMEM((_ACC_SIZE,), jnp.float32),
        pltpu.VMEM((_ACC_SIZE,), jnp.float32),
        pltpu.VMEM((_ACC_SIZE,), jnp.float32),
        pltpu.VMEM((_ACC_SIZE,), jnp.float32),
        pltpu.VMEM((_NSEG,), jnp.float32),
        pltpu.VMEM((_NSEG,), jnp.float32),
        pltpu.SemaphoreType.DMA,
        pltpu.SemaphoreType.DMA,
    ),
)(_sc_body)


def _tc_body(s_ref, c_ref, o_ref):
    s = jnp.sum(s_ref[...], axis=0, keepdims=True)
    c = jnp.sum(c_ref[...], axis=0, keepdims=True)
    nonempty = c > 0.0
    val = jnp.where(nonempty, jnp.abs(s) / jnp.where(nonempty, c, 1.0), 0.0)
    o_ref[...] = jnp.full((1, 1), jnp.sum(val) / jnp.float32(_NSEG))


def kernel(pred, target, input_img):
    x = input_img.reshape(24, 512, 512)
    p = pred.reshape(24, 512, 512)
    t = target.reshape(24, 512, 512)
    sums, cnts = _sc_call(x, p, t)
    loss = pl.pallas_call(
        _tc_body,
        out_shape=jax.ShapeDtypeStruct((1, 1), jnp.float32),
    )(sums, cnts)
    return loss[0, 0]
